# baseline (speedup 1.0000x reference)
import jax
import jax.numpy as jnp
from jax.experimental import pallas as pl


def _bias_add_kernel(h_ref, b_ref, o_ref):
    o_ref[...] = h_ref[...] + b_ref[...]


def _gat_layer(x, src, dst, edge_attr, p):
    W, a_src, a_dst, We, a_edge, b = p
    N = x.shape[0]
    h = x @ W
    alpha_src = h @ a_src
    alpha_dst = h @ a_dst
    alpha_edge = (edge_attr @ We) @ a_edge
    alpha = alpha_src[src] + alpha_dst[dst] + alpha_edge
    alpha = jax.nn.leaky_relu(alpha, 0.2)
    amax = jax.ops.segment_max(alpha, dst, num_segments=N)
    amax = jnp.where(jnp.isfinite(amax), amax, 0.0)
    ex = jnp.exp(alpha - amax[dst])
    denom = jax.ops.segment_sum(ex, dst, num_segments=N)
    coef = ex / (denom[dst] + 1e-16)
    msg = h[src] * coef[:, None]
    out = jax.ops.segment_sum(msg, dst, num_segments=N)
    bias = jnp.broadcast_to(b, out.shape)
    return pl.pallas_call(
        _bias_add_kernel,
        out_shape=jax.ShapeDtypeStruct(out.shape, out.dtype),
    )(out, bias)


def kernel(x, edge_index, edge_attr, params):
    src, dst = edge_index[0], edge_index[1]
    h = x
    for l in range(len(params) - 1):
        h = _gat_layer(h, src, dst, edge_attr, params[l])
        h = jax.nn.relu(h)
    return _gat_layer(h, src, dst, edge_attr, params[-1])


# speedup vs baseline: 8.9283x; 8.9283x over previous
import functools

import jax
import jax.numpy as jnp
from jax import lax
from jax.experimental import pallas as pl
from jax.experimental.pallas import tpu as pltpu
from jax.experimental.pallas import tpu_sc as plsc

N_NODES = 10000
NPAD = 10240
NT = 32
NPT = NPAD // NT
GARB = NPT
CHUNK = 512
RB = 32
E_PAD = 321536


def _mm_body(x_ref, w_ref, a8_ref, h_ref, aux_ref, *, relu):
    xb = x_ref[...]
    if relu:
        xb = jnp.maximum(xb, 0.0)
    h = jnp.dot(xb, w_ref[...], preferred_element_type=jnp.float32)
    h_ref[...] = h
    aux_ref[...] = lax.dot_general(
        a8_ref[...], h, (((1,), (1,)), ((), ())),
        preferred_element_type=jnp.float32)


def _mm(x, w, a8, relu):
    m, cin = x.shape
    cout = w.shape[1]
    bm = 1280
    return pl.pallas_call(
        functools.partial(_mm_body, relu=relu),
        grid=(m // bm,),
        in_specs=[
            pl.BlockSpec((bm, cin), lambda i: (i, 0)),
            pl.BlockSpec((cin, cout), lambda i: (0, 0)),
            pl.BlockSpec((8, cout), lambda i: (0, 0)),
        ],
        out_specs=[
            pl.BlockSpec((bm, cout), lambda i: (i, 0)),
            pl.BlockSpec((8, bm), lambda i: (0, i)),
        ],
        out_shape=[
            jax.ShapeDtypeStruct((m, cout), jnp.float32),
            jax.ShapeDtypeStruct((8, m), jnp.float32),
        ],
    )(x, w, a8)


def _ae_body(em_ref, ea_ref, o_ref):
    o_ref[...] = lax.dot_general(
        em_ref[...], ea_ref[...], (((0,), (1,)), ((), ())),
        preferred_element_type=jnp.float32)


def _ae_all(ea_s, emat):
    eb = E_PAD // 16
    return pl.pallas_call(
        _ae_body,
        grid=(16,),
        in_specs=[
            pl.BlockSpec((16, 16), lambda i: (0, 0)),
            pl.BlockSpec((eb, 16), lambda i: (i, 0)),
        ],
        out_specs=pl.BlockSpec((16, eb), lambda i: (0, i)),
        out_shape=jax.ShapeDtypeStruct((16, E_PAD), jnp.float32),
    )(emat, ea_s)


def _sc_gat(h, aux, ae_l, srcs, dsts, offs, b_pad, c):
    nv = c // 16
    mesh = plsc.VectorSubcoreMesh(core_axis_name="c", subcore_axis_name="s")

    @functools.partial(
        pl.kernel, mesh=mesh,
        compiler_params=pltpu.CompilerParams(needs_layout_passes=False),
        out_type=jax.ShapeDtypeStruct((NPAD, c), jnp.float32),
        scratch_types=[
            pltpu.VMEM((NPAD,), jnp.float32),
            pltpu.VMEM((NPAD,), jnp.float32),
            pltpu.VMEM((336,), jnp.float32),
            pltpu.VMEM((336,), jnp.float32),
            pltpu.VMEM((GARB + 1, c), jnp.float32),
            pltpu.VMEM((CHUNK,), jnp.int32),
            pltpu.VMEM((CHUNK,), jnp.int32),
            pltpu.VMEM((CHUNK,), jnp.float32),
            pltpu.VMEM((CHUNK + 16,), jnp.float32),
            pltpu.VMEM((CHUNK + 16,), jnp.int32),
            pltpu.VMEM((16,), jnp.int32),
            pltpu.VMEM((16,), jnp.float32),
            pltpu.VMEM((2, RB, c), jnp.float32),
            pltpu.VMEM((48,), jnp.int32),
            pltpu.VMEM((c,), jnp.float32),
            pltpu.SemaphoreType.DMA,
            pltpu.SemaphoreType.DMA,
        ],
    )
    def k(h_hbm, aux_hbm, ae_hbm, srcs_hbm, dsts_hbm, offs_hbm, b_hbm,
          out_hbm, asrc_v, adst_v, amax_v, den_v, out_v, srcb, dstb, aeb,
          coefb, lidxb, sdb, sxb, rowb, offs_v, b_v, sem0, sem1):
        wid = lax.axis_index("s") * 2 + lax.axis_index("c")
        lo = wid * NPT
        pltpu.sync_copy(offs_hbm, offs_v)
        pltpu.sync_copy(b_hbm, b_v)
        pltpu.sync_copy(aux_hbm.at[0], asrc_v)
        pltpu.sync_copy(aux_hbm.at[1], adst_v)
        ovec = offs_v[pl.ds(wid, 16)]
        start = ovec[0]
        end = ovec[1]
        astart = (start // 8) * 8
        cnt = end - astart
        nch = (cnt + (CHUNK - 1)) // CHUNK
        iot = lax.iota(jnp.int32, 16)

        def init_row(i, _):
            for cc in range(nv):
                out_v[i, pl.ds(cc * 16, 16)] = b_v[pl.ds(cc * 16, 16)]
            return 0
        lax.fori_loop(0, GARB + 1, init_row, 0)
        for i in range(336 // 16):
            amax_v[pl.ds(i * 16, 16)] = jnp.full((16,), -3e38, jnp.float32)
            den_v[pl.ds(i * 16, 16)] = jnp.zeros((16,), jnp.float32)

        def load_chunk(ch):
            off = astart + ch * CHUNK
            pltpu.sync_copy(srcs_hbm.at[pl.ds(off, CHUNK)], srcb)
            pltpu.sync_copy(dsts_hbm.at[pl.ds(off, CHUNK)], dstb)
            pltpu.sync_copy(ae_hbm.at[pl.ds(off, CHUNK)], aeb)

        def alpha_vec(v):
            s16 = srcb[pl.ds(v * 16, 16)]
            d16 = dstb[pl.ds(v * 16, 16)]
            ae16 = aeb[pl.ds(v * 16, 16)]
            al = (plsc.load_gather(asrc_v, [s16])
                  + plsc.load_gather(adst_v, [d16]) + ae16)
            al = jnp.where(al >= 0.0, al, al * jnp.float32(0.2))
            loc = d16 - lo
            valid = (loc >= 0) & (loc < NPT)
            lidx = jnp.where(valid, loc, GARB)
            return lidx, al

        def seg_scan(lidx, x, is_max):
            sdb[...] = lidx
            for kk in (1, 2, 4, 8):
                idxk = jnp.maximum(iot - kk, 0)
                sxb[...] = x
                pk = plsc.load_gather(sdb, [idxk])
                xk = plsc.load_gather(sxb, [idxk])
                comb = jnp.maximum(x, xk) if is_max else x + xk
                x = jnp.where((pk == lidx) & (iot >= kk), comb, x)
            nxt = plsc.load_gather(sdb, [jnp.minimum(iot + 1, 15)])
            is_last = (nxt != lidx) | (iot == 15)
            return x, is_last

        def p1_chunk(ch, _):
            load_chunk(ch)
            def body(v, __):
                lidx, al = alpha_vec(v)
                m, is_last = seg_scan(lidx, al, True)
                old = plsc.load_gather(amax_v, [lidx])
                plsc.store_scatter(amax_v, [lidx], jnp.maximum(old, m),
                                   mask=is_last)
                return 0
            lax.fori_loop(0, CHUNK // 16, body, 0)
            return 0
        lax.fori_loop(0, nch, p1_chunk, 0)

        def p2_chunk(ch, _):
            load_chunk(ch)
            def body(v, __):
                lidx, al = alpha_vec(v)
                ex = jnp.exp(al - plsc.load_gather(amax_v, [lidx]))
                s, is_last = seg_scan(lidx, ex, False)
                old = plsc.load_gather(den_v, [lidx])
                plsc.store_scatter(den_v, [lidx], old + s, mask=is_last)
                return 0
            lax.fori_loop(0, CHUNK // 16, body, 0)
            return 0
        lax.fori_loop(0, nch, p2_chunk, 0)

        def flush(cur, acc):
            for cc in range(nv):
                out_v[cur, pl.ds(cc * 16, 16)] = (
                    acc[cc] + b_v[pl.ds(cc * 16, 16)])

        def p3_chunk(ch, carry):
            load_chunk(ch)
            def cbody(v, __):
                lidx, al = alpha_vec(v)
                ex = jnp.exp(al - plsc.load_gather(amax_v, [lidx]))
                den = plsc.load_gather(den_v, [lidx])
                coefb[pl.ds(v * 16, 16)] = ex / (den + jnp.float32(1e-16))
                lidxb[pl.ds(v * 16, 16)] = lidx
                return 0
            lax.fori_loop(0, CHUNK // 16, cbody, 0)

            sems = (sem0, sem1)
            nsub = CHUNK // RB
            cps = [None] * nsub
            cps[0] = pltpu.async_copy(
                h_hbm.at[srcb.at[pl.ds(0, RB)]], rowb.at[0], sems[0])
            for j in range(nsub):
                if j + 1 < nsub:
                    cps[j + 1] = pltpu.async_copy(
                        h_hbm.at[srcb.at[pl.ds((j + 1) * RB, RB)]],
                        rowb.at[(j + 1) % 2], sems[(j + 1) % 2])
                cps[j].wait()
                buf = j % 2

                def ebody(i, ec):
                    cur = ec[0]
                    acc = ec[1:]
                    coef = coefb[pl.ds(j * RB + i, 16)][0]
                    li = lidxb[pl.ds(j * RB + i, 16)][0]
                    is_new = li != cur

                    @pl.when(is_new)
                    def _():
                        flush(cur, acc)

                    newacc = []
                    for cc in range(nv):
                        contrib = rowb[buf, i, pl.ds(cc * 16, 16)] * coef
                        newacc.append(
                            jnp.where(is_new, contrib, acc[cc] + contrib))
                    return (li,) + tuple(newacc)

                carry = lax.fori_loop(0, RB, ebody, carry)
            return carry

        carry0 = ((jnp.int32(GARB),)
                  + tuple(jnp.zeros((16,), jnp.float32) for _ in range(nv)))
        carry = lax.fori_loop(0, nch, p3_chunk, carry0)
        flush(carry[0], carry[1:])

        pltpu.sync_copy(out_v.at[pl.ds(0, NPT)], out_hbm.at[pl.ds(lo, NPT)])

    return k(h, aux, ae_l, srcs, dsts, offs, b_pad)


def kernel(x, edge_index, edge_attr, params):
    src = edge_index[0].astype(jnp.int32)
    dst = edge_index[1].astype(jnp.int32)
    e = src.shape[0]

    perm = jnp.argsort(dst)
    srcs = src[perm]
    dsts = dst[perm]
    eas = edge_attr[perm]

    srcs_p = jnp.concatenate(
        [srcs, jnp.zeros((E_PAD - e,), jnp.int32)])
    dsts_p = jnp.concatenate(
        [dsts, jnp.full((E_PAD - e,), NPAD, jnp.int32)])
    eas_p = jnp.concatenate(
        [eas, jnp.zeros((E_PAD - e, eas.shape[1]), jnp.float32)])
    offs = jnp.searchsorted(
        dsts, jnp.arange(NT + 1, dtype=jnp.int32) * NPT).astype(jnp.int32)
    offs_p = jnp.concatenate([offs, jnp.zeros((15,), jnp.int32)])

    emat = jnp.zeros((16, 16), jnp.float32)
    for l, p in enumerate(params):
        emat = emat.at[:, l].set(p[3] @ p[4])
    aet = _ae_all(eas_p, emat)

    hcur = jnp.pad(x, ((0, NPAD - x.shape[0]), (0, 0)))
    nlayers = len(params)
    for l, p in enumerate(params):
        w, a_src, a_dst, _, _, b = p
        cout = w.shape[1]
        cpad = max(128, cout)
        if cout < cpad:
            w = jnp.pad(w, ((0, 0), (0, cpad - cout)))
            a_src = jnp.pad(a_src, (0, cpad - cout))
            a_dst = jnp.pad(a_dst, (0, cpad - cout))
            b = jnp.pad(b, (0, cpad - cout))
        a8 = jnp.zeros((8, cpad), jnp.float32)
        a8 = a8.at[0].set(a_src).at[1].set(a_dst)
        h, aux = _mm(hcur, w, a8, relu=(l > 0))
        hcur = _sc_gat(h, aux, aet[l], srcs_p, dsts_p, offs_p, b, cpad)

    cout_final = params[-1][0].shape[1]
    return hcur[:N_NODES, :cout_final]


# speedup vs baseline: 18.9801x; 2.1258x over previous
import functools

import jax
import jax.numpy as jnp
from jax import lax
from jax.experimental import pallas as pl
from jax.experimental.pallas import tpu as pltpu
from jax.experimental.pallas import tpu_sc as plsc

N_NODES = 10000
NPAD = 10240
NT = 32
NPT = NPAD // NT
GARB = NPT
CHUNK = 256
RB = 64
E_PAD = 321536


def _mm_body(x_ref, w_ref, a8_ref, h_ref, aux_ref, *, relu):
    xb = x_ref[...]
    if relu:
        xb = jnp.maximum(xb, 0.0)
    h = jnp.dot(xb, w_ref[...], preferred_element_type=jnp.float32)
    h_ref[...] = h
    aux_ref[...] = lax.dot_general(
        a8_ref[...], h, (((1,), (1,)), ((), ())),
        preferred_element_type=jnp.float32)


def _mm(x, w, a8, relu):
    m, cin = x.shape
    cout = w.shape[1]
    bm = 1280
    return pl.pallas_call(
        functools.partial(_mm_body, relu=relu),
        grid=(m // bm,),
        in_specs=[
            pl.BlockSpec((bm, cin), lambda i: (i, 0)),
            pl.BlockSpec((cin, cout), lambda i: (0, 0)),
            pl.BlockSpec((8, cout), lambda i: (0, 0)),
        ],
        out_specs=[
            pl.BlockSpec((bm, cout), lambda i: (i, 0)),
            pl.BlockSpec((8, bm), lambda i: (0, i)),
        ],
        out_shape=[
            jax.ShapeDtypeStruct((m, cout), jnp.float32),
            jax.ShapeDtypeStruct((8, m), jnp.float32),
        ],
    )(x, w, a8)


def _ae_body(em_ref, ea_ref, o_ref):
    o_ref[...] = lax.dot_general(
        em_ref[...], ea_ref[...], (((0,), (1,)), ((), ())),
        preferred_element_type=jnp.float32)


def _ae_all(ea_s, emat):
    eb = E_PAD // 16
    return pl.pallas_call(
        _ae_body,
        grid=(16,),
        in_specs=[
            pl.BlockSpec((16, 16), lambda i: (0, 0)),
            pl.BlockSpec((eb, 16), lambda i: (i, 0)),
        ],
        out_specs=pl.BlockSpec((16, eb), lambda i: (0, i)),
        out_shape=jax.ShapeDtypeStruct((16, E_PAD), jnp.float32),
    )(emat, ea_s)


def _sc_gat(h, aux, ae_l, srcs, dsts, offs, b_pad, c):
    nv = c // 16
    mesh = plsc.VectorSubcoreMesh(core_axis_name="c", subcore_axis_name="s")

    @functools.partial(
        pl.kernel, mesh=mesh,
        compiler_params=pltpu.CompilerParams(needs_layout_passes=False),
        out_type=jax.ShapeDtypeStruct((NPAD, c), jnp.float32),
        scratch_types=[
            pltpu.VMEM((NPAD,), jnp.float32),
            pltpu.VMEM((448,), jnp.float32),
            pltpu.VMEM((336,), jnp.float32),
            pltpu.VMEM((336,), jnp.float32),
            pltpu.VMEM((GARB + 1, c), jnp.float32),
            pltpu.VMEM((CHUNK,), jnp.int32),
            pltpu.VMEM((CHUNK,), jnp.int32),
            pltpu.VMEM((CHUNK,), jnp.float32),
            pltpu.VMEM((CHUNK + 16,), jnp.float32),
            pltpu.VMEM((CHUNK + 16,), jnp.int32),
            pltpu.VMEM((16,), jnp.int32),
            pltpu.VMEM((16,), jnp.float32),
            pltpu.VMEM((2, RB, c), jnp.float32),
            pltpu.VMEM((48,), jnp.int32),
            pltpu.VMEM((c,), jnp.float32),
            pltpu.SemaphoreType.DMA,
            pltpu.SemaphoreType.DMA,
            pltpu.SemaphoreType.DMA,
        ],
    )
    def k(h_hbm, aux_hbm, ae_hbm, srcs_hbm, dsts_hbm, offs_hbm, b_hbm,
          out_hbm, asrc_v, adst_v, amax_v, den_v, out_v, srcb, dstb, aeb,
          coefb, lidxb, sdb, sxb, rowb, offs_v, b_v, sem0, sem1, semc):
        wid = lax.axis_index("s") * 2 + lax.axis_index("c")
        lo = wid * NPT
        pltpu.sync_copy(offs_hbm, offs_v)
        pltpu.sync_copy(b_hbm, b_v)
        pltpu.sync_copy(aux_hbm.at[0], asrc_v)
        albase = (lo // 128) * 128
        delta = lo - albase
        pltpu.sync_copy(aux_hbm.at[1].at[pl.ds(albase, 384)],
                        adst_v.at[pl.ds(0, 384)])
        for i in range(4):
            adst_v[pl.ds(384 + i * 16, 16)] = jnp.zeros((16,), jnp.float32)
        ovec = offs_v[pl.ds(wid, 16)]
        start = ovec[0]
        end = ovec[1]
        astart = (start // 8) * 8
        cnt = end - astart
        nch = (cnt + (CHUNK - 1)) // CHUNK
        iot = lax.iota(jnp.int32, 16)

        def init_row(i, _):
            for cc in range(nv):
                out_v[i, pl.ds(cc * 16, 16)] = jnp.zeros((16,), jnp.float32)
            return 0
        lax.fori_loop(0, GARB + 1, init_row, 0)
        for i in range(336 // 16):
            amax_v[pl.ds(i * 16, 16)] = jnp.full((16,), -3e38, jnp.float32)
            den_v[pl.ds(i * 16, 16)] = jnp.zeros((16,), jnp.float32)

        def load_chunk(ch):
            off = astart + ch * CHUNK
            c1 = pltpu.async_copy(srcs_hbm.at[pl.ds(off, CHUNK)], srcb, semc)
            c2 = pltpu.async_copy(dsts_hbm.at[pl.ds(off, CHUNK)], dstb, semc)
            c3 = pltpu.async_copy(ae_hbm.at[pl.ds(off, CHUNK)], aeb, semc)
            c1.wait()
            c2.wait()
            c3.wait()

        def alpha_vec(v):
            s16 = srcb[pl.ds(v * 16, 16)]
            d16 = dstb[pl.ds(v * 16, 16)]
            ae16 = aeb[pl.ds(v * 16, 16)]
            loc = d16 - lo
            valid = (loc >= 0) & (loc < NPT)
            lidx = jnp.where(valid, loc, GARB)
            al = (plsc.load_gather(asrc_v, [s16])
                  + plsc.load_gather(adst_v, [lidx + delta]) + ae16)
            al = jnp.where(al >= 0.0, al, al * jnp.float32(0.2))
            return lidx, al

        def seg_scan(lidx, x, is_max):
            sdb[...] = lidx
            for kk in (1, 2, 4, 8):
                idxk = jnp.maximum(iot - kk, 0)
                sxb[...] = x
                pk = plsc.load_gather(sdb, [idxk])
                xk = plsc.load_gather(sxb, [idxk])
                comb = jnp.maximum(x, xk) if is_max else x + xk
                x = jnp.where((pk == lidx) & (iot >= kk), comb, x)
            nxt = plsc.load_gather(sdb, [jnp.minimum(iot + 1, 15)])
            is_last = (nxt != lidx) | (iot == 15)
            return x, is_last

        def p1_chunk(ch, _):
            load_chunk(ch)
            def body(v, __):
                lidx, al = alpha_vec(v)
                m, is_last = seg_scan(lidx, al, True)
                old = plsc.load_gather(amax_v, [lidx])
                plsc.store_scatter(amax_v, [lidx], jnp.maximum(old, m),
                                   mask=is_last)
                return 0
            lax.fori_loop(0, CHUNK // 16, body, 0)
            return 0
        lax.fori_loop(0, nch, p1_chunk, 0)

        def flush(cur, acc):
            for cc in range(nv):
                out_v[cur, pl.ds(cc * 16, 16)] = acc[cc]

        def p3_chunk(ch, carry):
            load_chunk(ch)
            def cbody(v, __):
                lidx, al = alpha_vec(v)
                ex = jnp.exp(al - plsc.load_gather(amax_v, [lidx]))
                s, is_last = seg_scan(lidx, ex, False)
                old = plsc.load_gather(den_v, [lidx])
                plsc.store_scatter(den_v, [lidx], old + s, mask=is_last)
                coefb[pl.ds(v * 16, 16)] = ex
                lidxb[pl.ds(v * 16, 16)] = lidx
                return 0
            lax.fori_loop(0, CHUNK // 16, cbody, 0)

            sems = (sem0, sem1)
            nsub = CHUNK // RB
            cps = [None] * nsub
            cps[0] = pltpu.async_copy(
                h_hbm.at[srcb.at[pl.ds(0, RB)]], rowb.at[0], sems[0])
            for j in range(nsub):
                if j + 1 < nsub:
                    cps[j + 1] = pltpu.async_copy(
                        h_hbm.at[srcb.at[pl.ds((j + 1) * RB, RB)]],
                        rowb.at[(j + 1) % 2], sems[(j + 1) % 2])
                cps[j].wait()
                buf = j % 2

                def ebody(i, ec):
                    cur = ec[0]
                    acc = ec[1:]
                    coef = coefb[pl.ds(j * RB + i, 16)][0]
                    li = lidxb[pl.ds(j * RB + i, 16)][0]
                    is_new = li != cur

                    @pl.when(is_new)
                    def _():
                        flush(cur, acc)

                    newacc = []
                    for cc in range(nv):
                        contrib = rowb[buf, i, pl.ds(cc * 16, 16)] * coef
                        newacc.append(
                            jnp.where(is_new, contrib, acc[cc] + contrib))
                    return (li,) + tuple(newacc)

                carry = lax.fori_loop(0, RB, ebody, carry)
            return carry

        carry0 = ((jnp.int32(GARB),)
                  + tuple(jnp.zeros((16,), jnp.float32) for _ in range(nv)))
        carry = lax.fori_loop(0, nch, p3_chunk, carry0)
        flush(carry[0], carry[1:])

        def norm_row(r, _):
            d = den_v[pl.ds(r, 16)][0]
            dvec = jnp.zeros((16,), jnp.float32) + d
            inv = jnp.float32(1.0) / (dvec + jnp.float32(1e-16))
            for cc in range(nv):
                sl = pl.ds(cc * 16, 16)
                out_v[r, sl] = out_v[r, sl] * inv + b_v[sl]
            return 0
        lax.fori_loop(0, NPT, norm_row, 0)

        pltpu.sync_copy(out_v.at[pl.ds(0, NPT)], out_hbm.at[pl.ds(lo, NPT)])

    return k(h, aux, ae_l, srcs, dsts, offs, b_pad)


def kernel(x, edge_index, edge_attr, params):
    src = edge_index[0].astype(jnp.int32)
    dst = edge_index[1].astype(jnp.int32)
    e = src.shape[0]

    perm = jnp.argsort(dst)
    srcs = src[perm]
    dsts = dst[perm]
    eas = edge_attr[perm]

    srcs_p = jnp.concatenate(
        [srcs, jnp.zeros((E_PAD - e,), jnp.int32)])
    dsts_p = jnp.concatenate(
        [dsts, jnp.full((E_PAD - e,), NPAD, jnp.int32)])
    eas_p = jnp.concatenate(
        [eas, jnp.zeros((E_PAD - e, eas.shape[1]), jnp.float32)])
    offs = jnp.searchsorted(
        dsts, jnp.arange(NT + 1, dtype=jnp.int32) * NPT).astype(jnp.int32)
    offs_p = jnp.concatenate([offs, jnp.zeros((15,), jnp.int32)])

    emat = jnp.zeros((16, 16), jnp.float32)
    for l, p in enumerate(params):
        emat = emat.at[:, l].set(p[3] @ p[4])
    aet = _ae_all(eas_p, emat)

    hcur = jnp.pad(x, ((0, NPAD - x.shape[0]), (0, 0)))
    nlayers = len(params)
    for l, p in enumerate(params):
        w, a_src, a_dst, _, _, b = p
        cout = w.shape[1]
        cpad = max(128, cout)
        if cout < cpad:
            w = jnp.pad(w, ((0, 0), (0, cpad - cout)))
            a_src = jnp.pad(a_src, (0, cpad - cout))
            a_dst = jnp.pad(a_dst, (0, cpad - cout))
            b = jnp.pad(b, (0, cpad - cout))
        a8 = jnp.zeros((8, cpad), jnp.float32)
        a8 = a8.at[0].set(a_src).at[1].set(a_dst)
        h, aux = _mm(hcur, w, a8, relu=(l > 0))
        hcur = _sc_gat(h, aux, aet[l], srcs_p, dsts_p, offs_p, b, cpad)

    cout_final = params[-1][0].shape[1]
    return hcur[:N_NODES, :cout_final]


# speedup vs baseline: 19.5045x; 1.0276x over previous
import functools

import jax
import jax.numpy as jnp
from jax import lax
from jax.experimental import pallas as pl
from jax.experimental.pallas import tpu as pltpu
from jax.experimental.pallas import tpu_sc as plsc

N_NODES = 10000
NPAD = 10240
NT = 32
NPT = NPAD // NT
GARB = NPT
CHUNK = 256
RB = 64
E_PAD = 321536


def _mm_body(x_ref, w_ref, a8_ref, h_ref, aux_ref, *, relu):
    xb = x_ref[...]
    if relu:
        xb = jnp.maximum(xb, 0.0)
    h = jnp.dot(xb, w_ref[...], preferred_element_type=jnp.float32)
    h_ref[...] = h
    aux_ref[...] = lax.dot_general(
        a8_ref[...], h, (((1,), (1,)), ((), ())),
        preferred_element_type=jnp.float32)


def _mm(x, w, a8, relu):
    m, cin = x.shape
    cout = w.shape[1]
    bm = 1280
    return pl.pallas_call(
        functools.partial(_mm_body, relu=relu),
        grid=(m // bm,),
        in_specs=[
            pl.BlockSpec((bm, cin), lambda i: (i, 0)),
            pl.BlockSpec((cin, cout), lambda i: (0, 0)),
            pl.BlockSpec((8, cout), lambda i: (0, 0)),
        ],
        out_specs=[
            pl.BlockSpec((bm, cout), lambda i: (i, 0)),
            pl.BlockSpec((8, bm), lambda i: (0, i)),
        ],
        out_shape=[
            jax.ShapeDtypeStruct((m, cout), jnp.float32),
            jax.ShapeDtypeStruct((8, m), jnp.float32),
        ],
    )(x, w, a8)


def _ae_body(em_ref, ea_ref, o_ref):
    o_ref[...] = lax.dot_general(
        em_ref[...], ea_ref[...], (((0,), (1,)), ((), ())),
        preferred_element_type=jnp.float32)


def _ae_all(ea_s, emat):
    eb = E_PAD // 16
    return pl.pallas_call(
        _ae_body,
        grid=(16,),
        in_specs=[
            pl.BlockSpec((16, 16), lambda i: (0, 0)),
            pl.BlockSpec((eb, 16), lambda i: (i, 0)),
        ],
        out_specs=pl.BlockSpec((16, eb), lambda i: (0, i)),
        out_shape=jax.ShapeDtypeStruct((16, E_PAD), jnp.float32),
    )(emat, ea_s)


def _sc_gat(h, aux, ae_l, srcs, dsts, offs, b_pad, c):
    nv = c // 16
    mesh = plsc.VectorSubcoreMesh(core_axis_name="c", subcore_axis_name="s")

    @functools.partial(
        pl.kernel, mesh=mesh,
        compiler_params=pltpu.CompilerParams(needs_layout_passes=False),
        out_type=jax.ShapeDtypeStruct((NPAD, c), jnp.float32),
        scratch_types=[
            pltpu.VMEM((NPAD,), jnp.float32),
            pltpu.VMEM((448,), jnp.float32),
            pltpu.VMEM((336,), jnp.float32),
            pltpu.VMEM((336,), jnp.float32),
            pltpu.VMEM((GARB + 1, c), jnp.float32),
            pltpu.VMEM((CHUNK,), jnp.int32),
            pltpu.VMEM((CHUNK,), jnp.int32),
            pltpu.VMEM((CHUNK,), jnp.float32),
            pltpu.VMEM((CHUNK + 16,), jnp.float32),
            pltpu.VMEM((CHUNK + 16,), jnp.int32),
            pltpu.VMEM((16,), jnp.int32),
            pltpu.VMEM((16,), jnp.float32),
            pltpu.VMEM((2, RB, c), jnp.float32),
            pltpu.VMEM((48,), jnp.int32),
            pltpu.VMEM((c,), jnp.float32),
            pltpu.SemaphoreType.DMA,
            pltpu.SemaphoreType.DMA,
            pltpu.SemaphoreType.DMA,
        ],
    )
    def k(h_hbm, aux_hbm, ae_hbm, srcs_hbm, dsts_hbm, offs_hbm, b_hbm,
          out_hbm, asrc_v, adst_v, amax_v, den_v, out_v, srcb, dstb, aeb,
          coefb, lidxb, sdb, sxb, rowb, offs_v, b_v, sem0, sem1, semc):
        wid = lax.axis_index("s") * 2 + lax.axis_index("c")
        lo = wid * NPT
        pltpu.sync_copy(offs_hbm, offs_v)
        pltpu.sync_copy(b_hbm, b_v)
        pltpu.sync_copy(aux_hbm.at[0], asrc_v)
        albase = (lo // 128) * 128
        delta = lo - albase
        pltpu.sync_copy(aux_hbm.at[1].at[pl.ds(albase, 384)],
                        adst_v.at[pl.ds(0, 384)])
        for i in range(4):
            adst_v[pl.ds(384 + i * 16, 16)] = jnp.zeros((16,), jnp.float32)
        ovec = offs_v[pl.ds(wid, 16)]
        start = ovec[0]
        end = ovec[1]
        astart = (start // 8) * 8
        cnt = end - astart
        nch = (cnt + (CHUNK - 1)) // CHUNK
        iot = lax.iota(jnp.int32, 16)

        def init_row(i, _):
            for cc in range(nv):
                out_v[i, pl.ds(cc * 16, 16)] = jnp.zeros((16,), jnp.float32)
            return 0
        lax.fori_loop(0, GARB + 1, init_row, 0)
        for i in range(336 // 16):
            amax_v[pl.ds(i * 16, 16)] = jnp.full((16,), -3e38, jnp.float32)
            den_v[pl.ds(i * 16, 16)] = jnp.zeros((16,), jnp.float32)

        def load_chunk(ch):
            off = astart + ch * CHUNK
            c1 = pltpu.async_copy(srcs_hbm.at[pl.ds(off, CHUNK)], srcb, semc)
            c2 = pltpu.async_copy(dsts_hbm.at[pl.ds(off, CHUNK)], dstb, semc)
            c3 = pltpu.async_copy(ae_hbm.at[pl.ds(off, CHUNK)], aeb, semc)
            c1.wait()
            c2.wait()
            c3.wait()

        def alpha_vec(v):
            s16 = srcb[pl.ds(v * 16, 16)]
            d16 = dstb[pl.ds(v * 16, 16)]
            ae16 = aeb[pl.ds(v * 16, 16)]
            loc = d16 - lo
            valid = (loc >= 0) & (loc < NPT)
            lidx = jnp.where(valid, loc, GARB)
            al = (plsc.load_gather(asrc_v, [s16])
                  + plsc.load_gather(adst_v, [lidx + delta]) + ae16)
            al = jnp.where(al >= 0.0, al, al * jnp.float32(0.2))
            return lidx, al

        def seg_scan(lidx, x, is_max):
            sdb[...] = lidx
            for kk in (1, 2, 4, 8):
                idxk = jnp.maximum(iot - kk, 0)
                sxb[...] = x
                pk = plsc.load_gather(sdb, [idxk])
                xk = plsc.load_gather(sxb, [idxk])
                comb = jnp.maximum(x, xk) if is_max else x + xk
                x = jnp.where((pk == lidx) & (iot >= kk), comb, x)
            nxt = plsc.load_gather(sdb, [jnp.minimum(iot + 1, 15)])
            is_last = (nxt != lidx) | (iot == 15)
            return x, is_last

        def p1_chunk(ch, _):
            load_chunk(ch)
            def body(v, __):
                lidx, al = alpha_vec(v)
                m, is_last = seg_scan(lidx, al, True)
                old = plsc.load_gather(amax_v, [lidx])
                plsc.store_scatter(amax_v, [lidx], jnp.maximum(old, m),
                                   mask=is_last)
                return 0
            lax.fori_loop(0, CHUNK // 16, body, 0)
            return 0
        lax.fori_loop(0, nch, p1_chunk, 0)

        def flush(cur, acc):
            for cc in range(nv):
                out_v[cur, pl.ds(cc * 16, 16)] = acc[cc]

        def p3_chunk(ch, carry):
            load_chunk(ch)
            def cbody(v, __):
                lidx, al = alpha_vec(v)
                ex = jnp.exp(al - plsc.load_gather(amax_v, [lidx]))
                s, is_last = seg_scan(lidx, ex, False)
                old = plsc.load_gather(den_v, [lidx])
                plsc.store_scatter(den_v, [lidx], old + s, mask=is_last)
                coefb[pl.ds(v * 16, 16)] = ex
                lidxb[pl.ds(v * 16, 16)] = lidx
                return 0
            lax.fori_loop(0, CHUNK // 16, cbody, 0)

            sems = (sem0, sem1)
            nsub = CHUNK // RB
            cps = [None] * nsub
            cps[0] = pltpu.async_copy(
                h_hbm.at[srcb.at[pl.ds(0, RB)]], rowb.at[0], sems[0])
            for j in range(nsub):
                if j + 1 < nsub:
                    cps[j + 1] = pltpu.async_copy(
                        h_hbm.at[srcb.at[pl.ds((j + 1) * RB, RB)]],
                        rowb.at[(j + 1) % 2], sems[(j + 1) % 2])
                cps[j].wait()
                buf = j % 2

                def ebody(i, ec):
                    cur = ec[0]
                    acc = ec[1:]
                    coef = coefb[pl.ds(j * RB + i, 16)][0]
                    li = lidxb[pl.ds(j * RB + i, 16)][0]
                    is_new = li != cur

                    @pl.when(is_new)
                    def _():
                        flush(cur, acc)

                    keep = jnp.where(is_new, jnp.float32(0.0),
                                     jnp.float32(1.0))
                    newacc = []
                    for cc in range(nv):
                        contrib = rowb[buf, i, pl.ds(cc * 16, 16)] * coef
                        newacc.append(acc[cc] * keep + contrib)
                    return (li,) + tuple(newacc)

                carry = lax.fori_loop(0, RB, ebody, carry)
            return carry

        carry0 = ((jnp.int32(GARB),)
                  + tuple(jnp.zeros((16,), jnp.float32) for _ in range(nv)))
        carry = lax.fori_loop(0, nch, p3_chunk, carry0)
        flush(carry[0], carry[1:])

        def norm_row(r, _):
            d = den_v[pl.ds(r, 16)][0]
            dvec = jnp.zeros((16,), jnp.float32) + d
            inv = jnp.float32(1.0) / (dvec + jnp.float32(1e-16))
            for cc in range(nv):
                sl = pl.ds(cc * 16, 16)
                out_v[r, sl] = out_v[r, sl] * inv + b_v[sl]
            return 0
        lax.fori_loop(0, NPT, norm_row, 0)

        pltpu.sync_copy(out_v.at[pl.ds(0, NPT)], out_hbm.at[pl.ds(lo, NPT)])

    return k(h, aux, ae_l, srcs, dsts, offs, b_pad)


def kernel(x, edge_index, edge_attr, params):
    src = edge_index[0].astype(jnp.int32)
    dst = edge_index[1].astype(jnp.int32)
    e = src.shape[0]

    perm = jnp.argsort(dst)
    srcs = src[perm]
    dsts = dst[perm]
    eas = edge_attr[perm]

    srcs_p = jnp.concatenate(
        [srcs, jnp.zeros((E_PAD - e,), jnp.int32)])
    dsts_p = jnp.concatenate(
        [dsts, jnp.full((E_PAD - e,), NPAD, jnp.int32)])
    eas_p = jnp.concatenate(
        [eas, jnp.zeros((E_PAD - e, eas.shape[1]), jnp.float32)])
    offs = jnp.searchsorted(
        dsts, jnp.arange(NT + 1, dtype=jnp.int32) * NPT).astype(jnp.int32)
    offs_p = jnp.concatenate([offs, jnp.zeros((15,), jnp.int32)])

    emat = jnp.zeros((16, 16), jnp.float32)
    for l, p in enumerate(params):
        emat = emat.at[:, l].set(p[3] @ p[4])
    aet = _ae_all(eas_p, emat)

    hcur = jnp.pad(x, ((0, NPAD - x.shape[0]), (0, 0)))
    nlayers = len(params)
    for l, p in enumerate(params):
        w, a_src, a_dst, _, _, b = p
        cout = w.shape[1]
        cpad = max(128, cout)
        if cout < cpad:
            w = jnp.pad(w, ((0, 0), (0, cpad - cout)))
            a_src = jnp.pad(a_src, (0, cpad - cout))
            a_dst = jnp.pad(a_dst, (0, cpad - cout))
            b = jnp.pad(b, (0, cpad - cout))
        a8 = jnp.zeros((8, cpad), jnp.float32)
        a8 = a8.at[0].set(a_src).at[1].set(a_dst)
        h, aux = _mm(hcur, w, a8, relu=(l > 0))
        hcur = _sc_gat(h, aux, aet[l], srcs_p, dsts_p, offs_p, b, cpad)

    cout_final = params[-1][0].shape[1]
    return hcur[:N_NODES, :cout_final]


# speedup vs baseline: 19.5119x; 1.0004x over previous
import functools

import jax
import jax.numpy as jnp
from jax import lax
from jax.experimental import pallas as pl
from jax.experimental.pallas import tpu as pltpu
from jax.experimental.pallas import tpu_sc as plsc

N_NODES = 10000
NPAD = 10240
NT = 32
NPT = NPAD // NT
GARB = NPT
CHUNK = 256
RB = 64
E_PAD = 321536


def _mm_body(x_ref, w_ref, a8_ref, h_ref, aux_ref, *, relu):
    xb = x_ref[...]
    if relu:
        xb = jnp.maximum(xb, 0.0)
    h = jnp.dot(xb, w_ref[...], preferred_element_type=jnp.float32)
    h_ref[...] = h
    aux_ref[...] = lax.dot_general(
        a8_ref[...], h, (((1,), (1,)), ((), ())),
        preferred_element_type=jnp.float32)


def _mm(x, w, a8, relu):
    m, cin = x.shape
    cout = w.shape[1]
    bm = 1280
    return pl.pallas_call(
        functools.partial(_mm_body, relu=relu),
        grid=(m // bm,),
        in_specs=[
            pl.BlockSpec((bm, cin), lambda i: (i, 0)),
            pl.BlockSpec((cin, cout), lambda i: (0, 0)),
            pl.BlockSpec((8, cout), lambda i: (0, 0)),
        ],
        out_specs=[
            pl.BlockSpec((bm, cout), lambda i: (i, 0)),
            pl.BlockSpec((8, bm), lambda i: (0, i)),
        ],
        out_shape=[
            jax.ShapeDtypeStruct((m, cout), jnp.float32),
            jax.ShapeDtypeStruct((8, m), jnp.float32),
        ],
    )(x, w, a8)


def _ae_body(em_ref, ea_ref, o_ref):
    o_ref[...] = lax.dot_general(
        em_ref[...], ea_ref[...], (((0,), (1,)), ((), ())),
        preferred_element_type=jnp.float32)


def _ae_all(ea_s, emat):
    eb = E_PAD // 16
    return pl.pallas_call(
        _ae_body,
        grid=(16,),
        in_specs=[
            pl.BlockSpec((16, 16), lambda i: (0, 0)),
            pl.BlockSpec((eb, 16), lambda i: (i, 0)),
        ],
        out_specs=pl.BlockSpec((16, eb), lambda i: (0, i)),
        out_shape=jax.ShapeDtypeStruct((16, E_PAD), jnp.float32),
    )(emat, ea_s)


def _sc_gat(h, aux, ae_l, srcs, dsts, offs, b_pad, c):
    nv = c // 16
    mesh = plsc.VectorSubcoreMesh(core_axis_name="c", subcore_axis_name="s")

    @functools.partial(
        pl.kernel, mesh=mesh,
        compiler_params=pltpu.CompilerParams(needs_layout_passes=False),
        out_type=jax.ShapeDtypeStruct((NPAD, c), jnp.float32),
        scratch_types=[
            pltpu.VMEM((NPAD,), jnp.float32),
            pltpu.VMEM((448,), jnp.float32),
            pltpu.VMEM((336,), jnp.float32),
            pltpu.VMEM((336,), jnp.float32),
            pltpu.VMEM((GARB + 1, c), jnp.float32),
            pltpu.VMEM((CHUNK,), jnp.int32),
            pltpu.VMEM((CHUNK,), jnp.int32),
            pltpu.VMEM((CHUNK,), jnp.float32),
            pltpu.VMEM((CHUNK + 16,), jnp.float32),
            pltpu.VMEM((CHUNK + 16,), jnp.int32),
            pltpu.VMEM((16,), jnp.int32),
            pltpu.VMEM((16,), jnp.float32),
            pltpu.VMEM((2, RB, c), jnp.float32),
            pltpu.VMEM((48,), jnp.int32),
            pltpu.VMEM((c,), jnp.float32),
            pltpu.SemaphoreType.DMA,
            pltpu.SemaphoreType.DMA,
            pltpu.SemaphoreType.DMA,
        ],
    )
    def k(h_hbm, aux_hbm, ae_hbm, srcs_hbm, dsts_hbm, offs_hbm, b_hbm,
          out_hbm, asrc_v, adst_v, amax_v, den_v, out_v, srcb, dstb, aeb,
          coefb, lidxb, sdb, sxb, rowb, offs_v, b_v, sem0, sem1, semc):
        wid = lax.axis_index("s") * 2 + lax.axis_index("c")
        lo = wid * NPT
        pltpu.sync_copy(offs_hbm, offs_v)
        pltpu.sync_copy(b_hbm, b_v)
        pltpu.sync_copy(aux_hbm.at[0], asrc_v)
        albase = (lo // 128) * 128
        delta = lo - albase
        pltpu.sync_copy(aux_hbm.at[1].at[pl.ds(albase, 384)],
                        adst_v.at[pl.ds(0, 384)])
        for i in range(4):
            adst_v[pl.ds(384 + i * 16, 16)] = jnp.zeros((16,), jnp.float32)
        ovec = offs_v[pl.ds(wid, 16)]
        start = ovec[0]
        end = ovec[1]
        astart = (start // 8) * 8
        cnt = end - astart
        nch = (cnt + (CHUNK - 1)) // CHUNK
        iot = lax.iota(jnp.int32, 16)

        def init_row(i, _):
            for cc in range(nv):
                out_v[i, pl.ds(cc * 16, 16)] = jnp.zeros((16,), jnp.float32)
            return 0
        lax.fori_loop(0, GARB + 1, init_row, 0)
        for i in range(336 // 16):
            amax_v[pl.ds(i * 16, 16)] = jnp.full((16,), -3e38, jnp.float32)
            den_v[pl.ds(i * 16, 16)] = jnp.zeros((16,), jnp.float32)

        def load_chunk(ch):
            off = astart + ch * CHUNK
            c1 = pltpu.async_copy(srcs_hbm.at[pl.ds(off, CHUNK)], srcb, semc)
            c2 = pltpu.async_copy(dsts_hbm.at[pl.ds(off, CHUNK)], dstb, semc)
            c3 = pltpu.async_copy(ae_hbm.at[pl.ds(off, CHUNK)], aeb, semc)
            c1.wait()
            c2.wait()
            c3.wait()

        def alpha_vec(v):
            s16 = srcb[pl.ds(v * 16, 16)]
            d16 = dstb[pl.ds(v * 16, 16)]
            ae16 = aeb[pl.ds(v * 16, 16)]
            loc = d16 - lo
            valid = (loc >= 0) & (loc < NPT)
            lidx = jnp.where(valid, loc, GARB)
            al = (plsc.load_gather(asrc_v, [s16])
                  + plsc.load_gather(adst_v, [lidx + delta]) + ae16)
            al = jnp.where(al >= 0.0, al, al * jnp.float32(0.2))
            return lidx, al

        def seg_scan(lidx, x, is_max):
            sdb[...] = lidx
            for kk in (1, 2, 4, 8):
                idxk = jnp.maximum(iot - kk, 0)
                sxb[...] = x
                pk = plsc.load_gather(sdb, [idxk])
                xk = plsc.load_gather(sxb, [idxk])
                comb = jnp.maximum(x, xk) if is_max else x + xk
                x = jnp.where((pk == lidx) & (iot >= kk), comb, x)
            nxt = plsc.load_gather(sdb, [jnp.minimum(iot + 1, 15)])
            is_last = (nxt != lidx) | (iot == 15)
            return x, is_last

        def p1_chunk(ch, _):
            load_chunk(ch)
            def body(v, __):
                lidx, al = alpha_vec(v)
                m, is_last = seg_scan(lidx, al, True)
                old = plsc.load_gather(amax_v, [lidx])
                plsc.store_scatter(amax_v, [lidx], jnp.maximum(old, m),
                                   mask=is_last)
                return 0
            lax.fori_loop(0, CHUNK // 16, body, 0)
            return 0
        lax.fori_loop(0, nch, p1_chunk, 0)

        def flush(cur, acc):
            for cc in range(nv):
                out_v[cur, pl.ds(cc * 16, 16)] = acc[cc]

        def p3_chunk(ch, carry):
            load_chunk(ch)
            def cbody(v, __):
                lidx, al = alpha_vec(v)
                ex = jnp.exp(al - plsc.load_gather(amax_v, [lidx]))
                s, is_last = seg_scan(lidx, ex, False)
                old = plsc.load_gather(den_v, [lidx])
                plsc.store_scatter(den_v, [lidx], old + s, mask=is_last)
                coefb[pl.ds(v * 16, 16)] = ex
                lidxb[pl.ds(v * 16, 16)] = lidx
                return 0
            lax.fori_loop(0, CHUNK // 16, cbody, 0)

            sems = (sem0, sem1)
            nsub = CHUNK // RB
            cps = [None] * nsub
            cps[0] = pltpu.async_copy(
                h_hbm.at[srcb.at[pl.ds(0, RB)]], rowb.at[0], sems[0])
            for j in range(nsub):
                if j + 1 < nsub:
                    cps[j + 1] = pltpu.async_copy(
                        h_hbm.at[srcb.at[pl.ds((j + 1) * RB, RB)]],
                        rowb.at[(j + 1) % 2], sems[(j + 1) % 2])
                cps[j].wait()
                buf = j % 2

                def ebody(i, ec):
                    cur = ec[0]
                    acc = ec[1:]
                    coef = coefb[pl.ds(j * RB + i, 16)][0]
                    li = lidxb[pl.ds(j * RB + i, 16)][0]
                    is_new = li != cur

                    @pl.when(is_new)
                    def _():
                        flush(cur, acc)

                    keep = jnp.where(is_new, jnp.float32(0.0),
                                     jnp.float32(1.0))
                    newacc = []
                    for cc in range(nv):
                        contrib = rowb[buf, i, pl.ds(cc * 16, 16)] * coef
                        newacc.append(acc[cc] * keep + contrib)
                    return (li,) + tuple(newacc)

                carry = lax.fori_loop(0, RB, ebody, carry)
            return carry

        carry0 = ((jnp.int32(GARB),)
                  + tuple(jnp.zeros((16,), jnp.float32) for _ in range(nv)))
        carry = lax.fori_loop(0, nch, p3_chunk, carry0)
        flush(carry[0], carry[1:])

        def norm_row(r, _):
            d = den_v[pl.ds(r, 16)][0]
            dvec = jnp.zeros((16,), jnp.float32) + d
            inv = jnp.float32(1.0) / (dvec + jnp.float32(1e-16))
            for cc in range(nv):
                sl = pl.ds(cc * 16, 16)
                out_v[r, sl] = out_v[r, sl] * inv + b_v[sl]
            return 0
        lax.fori_loop(0, NPT, norm_row, 0)

        pltpu.sync_copy(out_v.at[pl.ds(0, NPT)], out_hbm.at[pl.ds(lo, NPT)])

    return k(h, aux, ae_l, srcs, dsts, offs, b_pad)


def kernel(x, edge_index, edge_attr, params):
    src = edge_index[0].astype(jnp.int32)
    dst = edge_index[1].astype(jnp.int32)
    e = src.shape[0]

    perm = jnp.argsort(dst)
    srcs = src[perm]
    dsts = dst[perm]
    eas = edge_attr[perm]

    srcs_p = jnp.concatenate(
        [srcs, jnp.zeros((E_PAD - e,), jnp.int32)])
    dsts_p = jnp.concatenate(
        [dsts, jnp.full((E_PAD - e,), NPAD, jnp.int32)])
    eas_p = jnp.concatenate(
        [eas, jnp.zeros((E_PAD - e, eas.shape[1]), jnp.float32)])
    offs = jnp.searchsorted(
        dsts, jnp.arange(NT + 1, dtype=jnp.int32) * NPT).astype(jnp.int32)
    offs_p = jnp.concatenate([offs, jnp.zeros((15,), jnp.int32)])

    emat = jnp.zeros((16, 16), jnp.float32)
    for l, p in enumerate(params):
        emat = emat.at[:, l].set(p[3] @ p[4])
    aet = _ae_all(eas_p, emat)

    hcur = jnp.pad(x, ((0, NPAD - x.shape[0]), (0, 0)))
    for l, p in enumerate(params):
        w, a_src, a_dst, _, _, b = p
        cout = w.shape[1]
        cpad = max(128, cout)
        if cout < cpad:
            w = jnp.pad(w, ((0, 0), (0, cpad - cout)))
            a_src = jnp.pad(a_src, (0, cpad - cout))
            a_dst = jnp.pad(a_dst, (0, cpad - cout))
            b = jnp.pad(b, (0, cpad - cout))
        a8 = jnp.zeros((8, cpad), jnp.float32)
        a8 = a8.at[0].set(a_src).at[1].set(a_dst)
        h, aux = _mm(hcur, w, a8, relu=(l > 0))
        hcur = _sc_gat(h, aux, aet[l], srcs_p, dsts_p, offs_p, b, cpad)

    cout_final = params[-1][0].shape[1]
    return hcur[:N_NODES, :cout_final]
